# baseline (speedup 1.0000x reference)
import functools

import jax
import jax.numpy as jnp
from jax import lax
from jax.experimental import pallas as pl
from jax.experimental.pallas import tpu as pltpu
from jax.experimental.pallas import tpu_sc as plsc

N = 10000
K = 32
NP = 10240
R = 80
C = 2048
TT = 80
TE = TT * K

_INF = 1e30
_BIGI = 2**30


def _bf16_rne(v):
    b = jax.lax.bitcast_convert_type(v, jnp.int32)
    r = (b + 0x7FFF + ((b >> 16) & 1)) & jnp.int32(-65536)
    return jax.lax.bitcast_convert_type(r, jnp.float32)


_M = 6


def _topk_body(rows_ref, cols_ref, idx_ref, xb2_scr, yb2_scr, sqc_scr):
    i = pl.program_id(0)
    xc_all = cols_ref[0:1, :]
    yc_all = cols_ref[1:2, :]
    xb2_scr[...] = -2.0 * _bf16_rne(xc_all)
    yb2_scr[...] = -2.0 * _bf16_rne(yc_all)
    sqc_scr[...] = xc_all * xc_all + yc_all * yc_all

    xr = rows_ref[:, 0:1]
    yr = rows_ref[:, 1:2]
    xrb = _bf16_rne(xr)
    yrb = _bf16_rne(yr)
    sqr = xr * xr + yr * yr
    row_gid = i * R + jax.lax.broadcasted_iota(jnp.int32, (R, 1), 0)
    lane32 = jax.lax.broadcasted_iota(jnp.int32, (R, K), 1)
    lane128 = jax.lax.broadcasted_iota(jnp.int32, (R, 128), 1)

    def d2_slice(start, width, lane_iota):
        xb2 = xb2_scr[0:1, pl.ds(start, width)]
        yb2 = yb2_scr[0:1, pl.ds(start, width)]
        sqc = sqc_scr[0:1, pl.ds(start, width)]
        d2 = (sqr + sqc) + (xrb * xb2 + yrb * yb2)
        cid = start + lane_iota
        return jnp.where(cid == row_gid, jnp.float32(1e10), d2), cid

    def extract_topk(a, ai):
        def extract(t4, ec):
            a, nv, ni = ec
            for u in range(8):
                t = t4 * 8 + u
                m = jnp.min(a, axis=1, keepdims=True)
                ci = jnp.min(jnp.where(a <= m, ai, _BIGI), axis=1, keepdims=True)
                a = jnp.where(ai == ci, _INF, a)
                nv = jnp.where(lane32 == t, m, nv)
                ni = jnp.where(lane32 == t, ci, ni)
            return a, nv, ni
        vals0 = jnp.full((R, K), _INF, jnp.float32)
        idxs0 = jnp.full((R, K), _BIGI, jnp.int32)
        _, nv, ni = jax.lax.fori_loop(0, K // 8, extract, (a, vals0, idxs0))
        return nv, ni

    def subtile_group(g, st):
        for u in range(16):
            s = g * 16 + u
            v = st[:_M]
            j = st[_M:]
            d2, cid = d2_slice(s * 128, 128, lane128)
            lt = [d2 < vm for vm in v]
            nv = [jnp.where(lt[0], d2, v[0])]
            nj = [jnp.where(lt[0], cid, j[0])]
            for m in range(1, _M):
                nv.append(jnp.where(lt[m - 1], v[m - 1], jnp.where(lt[m], d2, v[m])))
                nj.append(jnp.where(lt[m - 1], j[m - 1], jnp.where(lt[m], cid, j[m])))
            st = tuple(nv) + tuple(nj)
        return st

    init = (tuple(jnp.full((R, 128), _INF, jnp.float32) for _ in range(_M))
            + tuple(jnp.full((R, 128), _BIGI, jnp.int32) for _ in range(_M)))
    st = jax.lax.fori_loop(0, NP // 128 // 16, subtile_group, init)
    cand_v = jnp.concatenate(st[:_M - 1], axis=1)
    cand_i = jnp.concatenate(st[_M:2 * _M - 1], axis=1)
    nv, ni = extract_topk(cand_v, cand_i)
    tau = jnp.min(jnp.where(lane32 == K - 1, nv, _INF), axis=1, keepdims=True)
    unsafe = jnp.any(st[_M - 1] <= tau)

    def exact_path():
        laneC = jax.lax.broadcasted_iota(jnp.int32, (R, C), 1)

        def col_tile(c, carry):
            vals, idxs = carry
            d2, cid = d2_slice(c * C, C, laneC)
            a = jnp.concatenate([vals, d2], axis=1)
            ai = jnp.concatenate([idxs, cid], axis=1)
            def extract(t, ec):
                a, nv, ni = ec
                m = jnp.min(a, axis=1, keepdims=True)
                ci = jnp.min(jnp.where(a <= m, ai, _BIGI), axis=1, keepdims=True)
                a = jnp.where(ai == ci, _INF, a)
                nv = jnp.where(lane32 == t, m, nv)
                ni = jnp.where(lane32 == t, ci, ni)
                return a, nv, ni
            _, nv2, ni2 = jax.lax.fori_loop(0, K, extract, (a, vals, idxs))
            return nv2, ni2

        vals0 = jnp.full((R, K), _INF, jnp.float32)
        idxs0 = jnp.full((R, K), _BIGI, jnp.int32)
        _, idxs = jax.lax.fori_loop(0, NP // C, col_tile, (vals0, idxs0))
        return idxs

    idx_ref[...] = jax.lax.cond(unsafe, exact_path, lambda: ni)


def _mlp_body(gs_ref, gt_ref, w1t_ref, b1_ref, w2t_ref, b2_ref, out_ref):
    s4 = gs_ref[:, 4:5]
    s5 = gs_ref[:, 5:6]
    s6 = gs_ref[:, 6:7]
    s7 = gs_ref[:, 7:8]
    s8 = gs_ref[:, 8:9]
    s9 = gs_ref[:, 9:10]
    t4 = gt_ref[:, 4:5]
    t5 = gt_ref[:, 5:6]
    t6 = gt_ref[:, 6:7]
    t7 = gt_ref[:, 7:8]
    f1 = (s6 - t6) / s8
    f2 = (s7 - t7) / s9
    f3 = jnp.log(s4 / t4)
    f4 = jnp.log(s5 / t5)
    h = (f1 * w1t_ref[0:1, :] + f2 * w1t_ref[1:2, :]
         + f3 * w1t_ref[2:3, :] + f4 * w1t_ref[3:4, :]) + b1_ref[0:1, :]
    h = jnp.maximum(h, 0.0)
    o = jnp.dot(h, w2t_ref[...], preferred_element_type=jnp.float32)
    out_ref[...] = jnp.maximum(o + b2_ref[0:1, :], 0.0)


_E = N * K
_NW = 32
_BW = _E // _NW
_CHUNK = 2000
_NCH = _BW // _CHUNK


def _sc_gather(src, table):
    mesh = plsc.VectorSubcoreMesh(core_axis_name="c", subcore_axis_name="s")

    @functools.partial(
        pl.kernel, mesh=mesh,
        compiler_params=pltpu.CompilerParams(use_tc_tiling_on_sc=False),
        out_type=jax.ShapeDtypeStruct((_E, 16), jnp.float32),
        scratch_types=[
            pltpu.VMEM((_CHUNK,), jnp.int32),
            pltpu.VMEM((_CHUNK,), jnp.int32),
            pltpu.VMEM((_CHUNK, 16), jnp.float32),
            pltpu.VMEM((_CHUNK, 16), jnp.float32),
            pltpu.SemaphoreType.DMA,
            pltpu.SemaphoreType.DMA,
        ],
    )
    def gather_k(src_hbm, table_hbm, gs_hbm, idx_v0, idx_v1, rows_v0, rows_v1,
                 sem0, sem1):
        wid = lax.axis_index("s") * 2 + lax.axis_index("c")
        base = wid * _BW
        idx_v = (idx_v0, idx_v1)
        rows_v = (rows_v0, rows_v1)
        sem = (sem0, sem1)
        pending = [None] * 2
        for j in range(_NCH):
            b = j % 2
            pltpu.sync_copy(src_hbm.at[pl.ds(base + j * _CHUNK, _CHUNK)], idx_v[b])
            pending[b] = pltpu.async_copy(table_hbm.at[idx_v[b]], rows_v[b], sem[b])
            if j >= 1:
                pending[1 - b].wait()
                pltpu.sync_copy(rows_v[1 - b],
                                gs_hbm.at[pl.ds(base + (j - 1) * _CHUNK, _CHUNK)])
        pending[(_NCH - 1) % 2].wait()
        pltpu.sync_copy(rows_v[(_NCH - 1) % 2],
                        gs_hbm.at[pl.ds(base + (_NCH - 1) * _CHUNK, _CHUNK)])

    return gather_k(src, table)


def _knn_topk(pos):
    pos_t = jnp.transpose(pos)
    pad = jnp.full((2, NP - N), 1e4, jnp.float32)
    cols = jnp.concatenate([pos_t, pad], axis=1)
    return pl.pallas_call(
        _topk_body,
        grid=(N // R,),
        in_specs=[
            pl.BlockSpec((R, 2), lambda i: (i, 0)),
            pl.BlockSpec((2, NP), lambda i: (0, 0)),
        ],
        out_specs=pl.BlockSpec((R, K), lambda i: (i, 0)),
        out_shape=jax.ShapeDtypeStruct((N, K), jnp.int32),
        scratch_shapes=[
            pltpu.VMEM((1, NP), jnp.float32),
            pltpu.VMEM((1, NP), jnp.float32),
            pltpu.VMEM((1, NP), jnp.float32),
        ],
    )(pos, cols)


def _edge_mlp(gs, gt, w1t, b1, w2t, b2):
    e = gs.shape[0]
    return pl.pallas_call(
        _mlp_body,
        grid=(e // TE,),
        in_specs=[
            pl.BlockSpec((TE, 16), lambda i: (i, 0)),
            pl.BlockSpec((TE, 16), lambda i: (i, 0)),
            pl.BlockSpec((8, 64), lambda i: (0, 0)),
            pl.BlockSpec((1, 64), lambda i: (0, 0)),
            pl.BlockSpec((64, 128), lambda i: (0, 0)),
            pl.BlockSpec((1, 128), lambda i: (0, 0)),
        ],
        out_specs=pl.BlockSpec((TE, 128), lambda i: (i, 0)),
        out_shape=jax.ShapeDtypeStruct((e, 128), jnp.float32),
    )(gs, gt, w1t, b1, w2t, b2)


def kernel(x, location_info, W1, b1, W2, b2, k):
    del x, k
    li = location_info
    pos = li[:, 6:8]
    idx = _knn_topk(pos)
    src = idx.reshape(-1)
    tgt = jnp.repeat(jnp.arange(N, dtype=jnp.int32), K)
    edge_index = jnp.stack([src, tgt], axis=0)

    li16 = jnp.pad(li, ((0, 0), (0, 6)))
    gs = _sc_gather(src, li16)
    gt = jnp.repeat(li16, K, axis=0)

    w1t = jnp.pad(jnp.transpose(W1), ((0, 4), (0, 0)))
    w2t = jnp.transpose(W2)
    edge_attr = _edge_mlp(gs, gt, w1t, b1.reshape(1, -1), w2t, b2.reshape(1, -1))
    return edge_index, edge_attr


# speedup vs baseline: 1.2989x; 1.2989x over previous
import functools

import jax
import jax.numpy as jnp
from jax import lax
from jax.experimental import pallas as pl
from jax.experimental.pallas import tpu as pltpu
from jax.experimental.pallas import tpu_sc as plsc

N = 10000
K = 32
NP = 10240
R = 200
C = 2048
TT = 80
TE = TT * K

_INF = 1e30
_BIGI = 2**30


def _bf16_rne(v):
    b = jax.lax.bitcast_convert_type(v, jnp.int32)
    r = (b + 0x7FFF + ((b >> 16) & 1)) & jnp.int32(-65536)
    return jax.lax.bitcast_convert_type(r, jnp.float32)


_M = 6


def _topk_body(rows_ref, cols_ref, idx_ref, xb2_scr, yb2_scr, sqc_scr):
    i = pl.program_id(0)
    xc_all = cols_ref[0:1, :]
    yc_all = cols_ref[1:2, :]
    xb2_scr[...] = -2.0 * _bf16_rne(xc_all)
    yb2_scr[...] = -2.0 * _bf16_rne(yc_all)
    sqc_scr[...] = xc_all * xc_all + yc_all * yc_all

    xr = rows_ref[:, 0:1]
    yr = rows_ref[:, 1:2]
    xrb = _bf16_rne(xr)
    yrb = _bf16_rne(yr)
    sqr = xr * xr + yr * yr
    row_gid = i * R + jax.lax.broadcasted_iota(jnp.int32, (R, 1), 0)
    lane32 = jax.lax.broadcasted_iota(jnp.int32, (R, K), 1)
    lane128 = jax.lax.broadcasted_iota(jnp.int32, (R, 128), 1)

    def d2_slice(start, width, lane_iota):
        xb2 = xb2_scr[0:1, pl.ds(start, width)]
        yb2 = yb2_scr[0:1, pl.ds(start, width)]
        sqc = sqc_scr[0:1, pl.ds(start, width)]
        d2 = (sqr + sqc) + (xrb * xb2 + yrb * yb2)
        cid = start + lane_iota
        return jnp.where(cid == row_gid, jnp.float32(1e10), d2), cid

    def extract_topk(a, ai):
        def extract(t4, ec):
            a, nv, ni = ec
            for u in range(8):
                t = t4 * 8 + u
                m = jnp.min(a, axis=1, keepdims=True)
                ci = jnp.min(jnp.where(a <= m, ai, _BIGI), axis=1, keepdims=True)
                a = jnp.where(ai == ci, _INF, a)
                nv = jnp.where(lane32 == t, m, nv)
                ni = jnp.where(lane32 == t, ci, ni)
            return a, nv, ni
        vals0 = jnp.full((R, K), _INF, jnp.float32)
        idxs0 = jnp.full((R, K), _BIGI, jnp.int32)
        _, nv, ni = jax.lax.fori_loop(0, K // 8, extract, (a, vals0, idxs0))
        return nv, ni

    def subtile_group(g, st):
        for u in range(16):
            s = g * 16 + u
            v = st[:_M]
            j = st[_M:]
            d2, cid = d2_slice(s * 128, 128, lane128)
            lt = [d2 < vm for vm in v]
            nv = [jnp.where(lt[0], d2, v[0])]
            nj = [jnp.where(lt[0], cid, j[0])]
            for m in range(1, _M):
                nv.append(jnp.where(lt[m - 1], v[m - 1], jnp.where(lt[m], d2, v[m])))
                nj.append(jnp.where(lt[m - 1], j[m - 1], jnp.where(lt[m], cid, j[m])))
            st = tuple(nv) + tuple(nj)
        return st

    init = (tuple(jnp.full((R, 128), _INF, jnp.float32) for _ in range(_M))
            + tuple(jnp.full((R, 128), _BIGI, jnp.int32) for _ in range(_M)))
    st = jax.lax.fori_loop(0, NP // 128 // 16, subtile_group, init)
    cand_v = jnp.concatenate(st[:_M - 1], axis=1)
    cand_i = jnp.concatenate(st[_M:2 * _M - 1], axis=1)
    nv, ni = extract_topk(cand_v, cand_i)
    tau = jnp.min(jnp.where(lane32 == K - 1, nv, _INF), axis=1, keepdims=True)
    unsafe = jnp.any(st[_M - 1] <= tau)

    def exact_path():
        laneC = jax.lax.broadcasted_iota(jnp.int32, (R, C), 1)

        def col_tile(c, carry):
            vals, idxs = carry
            d2, cid = d2_slice(c * C, C, laneC)
            a = jnp.concatenate([vals, d2], axis=1)
            ai = jnp.concatenate([idxs, cid], axis=1)
            def extract(t, ec):
                a, nv, ni = ec
                m = jnp.min(a, axis=1, keepdims=True)
                ci = jnp.min(jnp.where(a <= m, ai, _BIGI), axis=1, keepdims=True)
                a = jnp.where(ai == ci, _INF, a)
                nv = jnp.where(lane32 == t, m, nv)
                ni = jnp.where(lane32 == t, ci, ni)
                return a, nv, ni
            _, nv2, ni2 = jax.lax.fori_loop(0, K, extract, (a, vals, idxs))
            return nv2, ni2

        vals0 = jnp.full((R, K), _INF, jnp.float32)
        idxs0 = jnp.full((R, K), _BIGI, jnp.int32)
        _, idxs = jax.lax.fori_loop(0, NP // C, col_tile, (vals0, idxs0))
        return idxs

    idx_ref[...] = jax.lax.cond(unsafe, exact_path, lambda: ni)


def _mlp_body(gs_ref, gt_ref, w1t_ref, b1_ref, w2t_ref, b2_ref, out_ref):
    s4 = gs_ref[:, 4:5]
    s5 = gs_ref[:, 5:6]
    s6 = gs_ref[:, 6:7]
    s7 = gs_ref[:, 7:8]
    s8 = gs_ref[:, 8:9]
    s9 = gs_ref[:, 9:10]
    t4 = gt_ref[:, 4:5]
    t5 = gt_ref[:, 5:6]
    t6 = gt_ref[:, 6:7]
    t7 = gt_ref[:, 7:8]
    f1 = (s6 - t6) / s8
    f2 = (s7 - t7) / s9
    f3 = jnp.log(s4 / t4)
    f4 = jnp.log(s5 / t5)
    h = (f1 * w1t_ref[0:1, :] + f2 * w1t_ref[1:2, :]
         + f3 * w1t_ref[2:3, :] + f4 * w1t_ref[3:4, :]) + b1_ref[0:1, :]
    h = jnp.maximum(h, 0.0)
    o = jnp.dot(h, w2t_ref[...], preferred_element_type=jnp.float32)
    out_ref[...] = jnp.maximum(o + b2_ref[0:1, :], 0.0)


_E = N * K
_NW = 32
_BW = _E // _NW
_CHUNK = 2000
_NCH = _BW // _CHUNK


def _sc_gather(src, table):
    mesh = plsc.VectorSubcoreMesh(core_axis_name="c", subcore_axis_name="s")

    @functools.partial(
        pl.kernel, mesh=mesh,
        compiler_params=pltpu.CompilerParams(use_tc_tiling_on_sc=False),
        out_type=jax.ShapeDtypeStruct((_E, 16), jnp.float32),
        scratch_types=[
            pltpu.VMEM((_CHUNK,), jnp.int32),
            pltpu.VMEM((_CHUNK,), jnp.int32),
            pltpu.VMEM((_CHUNK, 16), jnp.float32),
            pltpu.VMEM((_CHUNK, 16), jnp.float32),
            pltpu.SemaphoreType.DMA,
            pltpu.SemaphoreType.DMA,
        ],
    )
    def gather_k(src_hbm, table_hbm, gs_hbm, idx_v0, idx_v1, rows_v0, rows_v1,
                 sem0, sem1):
        wid = lax.axis_index("s") * 2 + lax.axis_index("c")
        base = wid * _BW
        idx_v = (idx_v0, idx_v1)
        rows_v = (rows_v0, rows_v1)
        sem = (sem0, sem1)
        pending = [None] * 2
        for j in range(_NCH):
            b = j % 2
            pltpu.sync_copy(src_hbm.at[pl.ds(base + j * _CHUNK, _CHUNK)], idx_v[b])
            pending[b] = pltpu.async_copy(table_hbm.at[idx_v[b]], rows_v[b], sem[b])
            if j >= 1:
                pending[1 - b].wait()
                pltpu.sync_copy(rows_v[1 - b],
                                gs_hbm.at[pl.ds(base + (j - 1) * _CHUNK, _CHUNK)])
        pending[(_NCH - 1) % 2].wait()
        pltpu.sync_copy(rows_v[(_NCH - 1) % 2],
                        gs_hbm.at[pl.ds(base + (_NCH - 1) * _CHUNK, _CHUNK)])

    return gather_k(src, table)


def _knn_topk(pos):
    pos_t = jnp.transpose(pos)
    pad = jnp.full((2, NP - N), 1e4, jnp.float32)
    cols = jnp.concatenate([pos_t, pad], axis=1)
    return pl.pallas_call(
        _topk_body,
        grid=(N // R,),
        in_specs=[
            pl.BlockSpec((R, 2), lambda i: (i, 0)),
            pl.BlockSpec((2, NP), lambda i: (0, 0)),
        ],
        out_specs=pl.BlockSpec((R, K), lambda i: (i, 0)),
        out_shape=jax.ShapeDtypeStruct((N, K), jnp.int32),
        scratch_shapes=[
            pltpu.VMEM((1, NP), jnp.float32),
            pltpu.VMEM((1, NP), jnp.float32),
            pltpu.VMEM((1, NP), jnp.float32),
        ],
    )(pos, cols)


def _edge_mlp(gs, gt, w1t, b1, w2t, b2):
    e = gs.shape[0]
    return pl.pallas_call(
        _mlp_body,
        grid=(e // TE,),
        in_specs=[
            pl.BlockSpec((TE, 16), lambda i: (i, 0)),
            pl.BlockSpec((TE, 16), lambda i: (i, 0)),
            pl.BlockSpec((8, 64), lambda i: (0, 0)),
            pl.BlockSpec((1, 64), lambda i: (0, 0)),
            pl.BlockSpec((64, 128), lambda i: (0, 0)),
            pl.BlockSpec((1, 128), lambda i: (0, 0)),
        ],
        out_specs=pl.BlockSpec((TE, 128), lambda i: (i, 0)),
        out_shape=jax.ShapeDtypeStruct((e, 128), jnp.float32),
    )(gs, gt, w1t, b1, w2t, b2)


def kernel(x, location_info, W1, b1, W2, b2, k):
    del x, k
    li = location_info
    pos = li[:, 6:8]
    idx = _knn_topk(pos)
    src = idx.reshape(-1)
    tgt = jnp.repeat(jnp.arange(N, dtype=jnp.int32), K)
    edge_index = jnp.stack([src, tgt], axis=0)

    li16 = jnp.pad(li, ((0, 0), (0, 6)))
    gs = _sc_gather(src, li16)
    gt = jnp.repeat(li16, K, axis=0)

    w1t = jnp.pad(jnp.transpose(W1), ((0, 4), (0, 0)))
    w2t = jnp.transpose(W2)
    edge_attr = _edge_mlp(gs, gt, w1t, b1.reshape(1, -1), w2t, b2.reshape(1, -1))
    return edge_index, edge_attr


# speedup vs baseline: 1.3227x; 1.0184x over previous
import functools

import jax
import jax.numpy as jnp
from jax import lax
from jax.experimental import pallas as pl
from jax.experimental.pallas import tpu as pltpu
from jax.experimental.pallas import tpu_sc as plsc

N = 10000
K = 32
NP = 10240
R = 200
C = 2048
TT = 200
TE = TT * K

_INF = 1e30
_BIGI = 2**30


def _bf16_rne(v):
    b = jax.lax.bitcast_convert_type(v, jnp.int32)
    r = (b + 0x7FFF + ((b >> 16) & 1)) & jnp.int32(-65536)
    return jax.lax.bitcast_convert_type(r, jnp.float32)


_M = 6


def _topk_body(rows_ref, cols_ref, idx_ref, xb2_scr, yb2_scr, sqc_scr):
    i = pl.program_id(0)
    xc_all = cols_ref[0:1, :]
    yc_all = cols_ref[1:2, :]
    xb2_scr[...] = -2.0 * _bf16_rne(xc_all)
    yb2_scr[...] = -2.0 * _bf16_rne(yc_all)
    sqc_scr[...] = xc_all * xc_all + yc_all * yc_all

    xr = rows_ref[:, 0:1]
    yr = rows_ref[:, 1:2]
    xrb = _bf16_rne(xr)
    yrb = _bf16_rne(yr)
    sqr = xr * xr + yr * yr
    row_gid = i * R + jax.lax.broadcasted_iota(jnp.int32, (R, 1), 0)
    lane32 = jax.lax.broadcasted_iota(jnp.int32, (R, K), 1)
    lane128 = jax.lax.broadcasted_iota(jnp.int32, (R, 128), 1)

    def d2_slice(start, width, lane_iota):
        xb2 = xb2_scr[0:1, pl.ds(start, width)]
        yb2 = yb2_scr[0:1, pl.ds(start, width)]
        sqc = sqc_scr[0:1, pl.ds(start, width)]
        d2 = (sqr + sqc) + (xrb * xb2 + yrb * yb2)
        cid = start + lane_iota
        return jnp.where(cid == row_gid, jnp.float32(1e10), d2), cid

    def extract_topk(a, ai):
        def extract(t4, ec):
            a, nv, ni = ec
            for u in range(8):
                t = t4 * 8 + u
                m = jnp.min(a, axis=1, keepdims=True)
                ci = jnp.min(jnp.where(a <= m, ai, _BIGI), axis=1, keepdims=True)
                a = jnp.where(ai == ci, _INF, a)
                nv = jnp.where(lane32 == t, m, nv)
                ni = jnp.where(lane32 == t, ci, ni)
            return a, nv, ni
        vals0 = jnp.full((R, K), _INF, jnp.float32)
        idxs0 = jnp.full((R, K), _BIGI, jnp.int32)
        _, nv, ni = jax.lax.fori_loop(0, K // 8, extract, (a, vals0, idxs0))
        return nv, ni

    def subtile_group(g, st):
        for u in range(16):
            s = g * 16 + u
            v = st[:_M]
            j = st[_M:]
            d2, cid = d2_slice(s * 128, 128, lane128)
            lt = [d2 < vm for vm in v]
            nv = [jnp.where(lt[0], d2, v[0])]
            nj = [jnp.where(lt[0], cid, j[0])]
            for m in range(1, _M):
                nv.append(jnp.where(lt[m - 1], v[m - 1], jnp.where(lt[m], d2, v[m])))
                nj.append(jnp.where(lt[m - 1], j[m - 1], jnp.where(lt[m], cid, j[m])))
            st = tuple(nv) + tuple(nj)
        return st

    init = (tuple(jnp.full((R, 128), _INF, jnp.float32) for _ in range(_M))
            + tuple(jnp.full((R, 128), _BIGI, jnp.int32) for _ in range(_M)))
    st = jax.lax.fori_loop(0, NP // 128 // 16, subtile_group, init)
    cand_v = jnp.concatenate(st[:_M - 1], axis=1)
    cand_i = jnp.concatenate(st[_M:2 * _M - 1], axis=1)
    nv, ni = extract_topk(cand_v, cand_i)
    tau = jnp.min(jnp.where(lane32 == K - 1, nv, _INF), axis=1, keepdims=True)
    unsafe = jnp.any(st[_M - 1] <= tau)

    def exact_path():
        laneC = jax.lax.broadcasted_iota(jnp.int32, (R, C), 1)

        def col_tile(c, carry):
            vals, idxs = carry
            d2, cid = d2_slice(c * C, C, laneC)
            a = jnp.concatenate([vals, d2], axis=1)
            ai = jnp.concatenate([idxs, cid], axis=1)
            def extract(t, ec):
                a, nv, ni = ec
                m = jnp.min(a, axis=1, keepdims=True)
                ci = jnp.min(jnp.where(a <= m, ai, _BIGI), axis=1, keepdims=True)
                a = jnp.where(ai == ci, _INF, a)
                nv = jnp.where(lane32 == t, m, nv)
                ni = jnp.where(lane32 == t, ci, ni)
                return a, nv, ni
            _, nv2, ni2 = jax.lax.fori_loop(0, K, extract, (a, vals, idxs))
            return nv2, ni2

        vals0 = jnp.full((R, K), _INF, jnp.float32)
        idxs0 = jnp.full((R, K), _BIGI, jnp.int32)
        _, idxs = jax.lax.fori_loop(0, NP // C, col_tile, (vals0, idxs0))
        return idxs

    idx_ref[...] = jax.lax.cond(unsafe, exact_path, lambda: ni)


def _mlp_body(gs_ref, gt_ref, w1t_ref, b1_ref, w2t_ref, b2_ref, out_ref):
    s4 = gs_ref[:, 4:5]
    s5 = gs_ref[:, 5:6]
    s6 = gs_ref[:, 6:7]
    s7 = gs_ref[:, 7:8]
    s8 = gs_ref[:, 8:9]
    s9 = gs_ref[:, 9:10]
    t4 = gt_ref[:, 4:5]
    t5 = gt_ref[:, 5:6]
    t6 = gt_ref[:, 6:7]
    t7 = gt_ref[:, 7:8]
    f1 = (s6 - t6) / s8
    f2 = (s7 - t7) / s9
    f3 = jnp.log(s4 / t4)
    f4 = jnp.log(s5 / t5)
    h = (f1 * w1t_ref[0:1, :] + f2 * w1t_ref[1:2, :]
         + f3 * w1t_ref[2:3, :] + f4 * w1t_ref[3:4, :]) + b1_ref[0:1, :]
    h = jnp.maximum(h, 0.0)
    o = jnp.dot(h, w2t_ref[...], preferred_element_type=jnp.float32)
    out_ref[...] = jnp.maximum(o + b2_ref[0:1, :], 0.0)


_E = N * K
_NW = 32
_BW = _E // _NW
_CHUNK = 2000
_NCH = _BW // _CHUNK


def _sc_gather(src, table):
    mesh = plsc.VectorSubcoreMesh(core_axis_name="c", subcore_axis_name="s")

    @functools.partial(
        pl.kernel, mesh=mesh,
        compiler_params=pltpu.CompilerParams(use_tc_tiling_on_sc=False),
        out_type=jax.ShapeDtypeStruct((_E, 16), jnp.float32),
        scratch_types=[
            pltpu.VMEM((_CHUNK,), jnp.int32),
            pltpu.VMEM((_CHUNK,), jnp.int32),
            pltpu.VMEM((_CHUNK, 16), jnp.float32),
            pltpu.VMEM((_CHUNK, 16), jnp.float32),
            pltpu.SemaphoreType.DMA,
            pltpu.SemaphoreType.DMA,
            pltpu.SemaphoreType.DMA,
            pltpu.SemaphoreType.DMA,
            pltpu.SemaphoreType.DMA,
            pltpu.SemaphoreType.DMA,
        ],
    )
    def gather_k(src_hbm, table_hbm, gs_hbm, idx_v0, idx_v1, rows_v0, rows_v1,
                 si0, si1, sg0, sg1, so0, so1):
        wid = lax.axis_index("s") * 2 + lax.axis_index("c")
        base = wid * _BW
        idx_v = (idx_v0, idx_v1)
        rows_v = (rows_v0, rows_v1)
        si = (si0, si1)
        sg = (sg0, sg1)
        so = (so0, so1)
        idx_pend = [None] * 2
        out_pend = [None] * 2
        idx_pend[0] = pltpu.async_copy(
            src_hbm.at[pl.ds(base, _CHUNK)], idx_v[0], si[0])
        for j in range(_NCH):
            b = j % 2
            nb = 1 - b
            if j + 1 < _NCH:
                idx_pend[nb] = pltpu.async_copy(
                    src_hbm.at[pl.ds(base + (j + 1) * _CHUNK, _CHUNK)],
                    idx_v[nb], si[nb])
            idx_pend[b].wait()
            if out_pend[b] is not None:
                out_pend[b].wait()
            pltpu.async_copy(table_hbm.at[idx_v[b]], rows_v[b], sg[b]).wait()
            out_pend[b] = pltpu.async_copy(
                rows_v[b], gs_hbm.at[pl.ds(base + j * _CHUNK, _CHUNK)], so[b])
        for p in out_pend:
            if p is not None:
                p.wait()

    return gather_k(src, table)


def _knn_topk(pos):
    pos_t = jnp.transpose(pos)
    pad = jnp.full((2, NP - N), 1e4, jnp.float32)
    cols = jnp.concatenate([pos_t, pad], axis=1)
    return pl.pallas_call(
        _topk_body,
        grid=(N // R,),
        in_specs=[
            pl.BlockSpec((R, 2), lambda i: (i, 0)),
            pl.BlockSpec((2, NP), lambda i: (0, 0)),
        ],
        out_specs=pl.BlockSpec((R, K), lambda i: (i, 0)),
        out_shape=jax.ShapeDtypeStruct((N, K), jnp.int32),
        scratch_shapes=[
            pltpu.VMEM((1, NP), jnp.float32),
            pltpu.VMEM((1, NP), jnp.float32),
            pltpu.VMEM((1, NP), jnp.float32),
        ],
    )(pos, cols)


def _edge_mlp(gs, gt, w1t, b1, w2t, b2):
    e = gs.shape[0]
    return pl.pallas_call(
        _mlp_body,
        grid=(e // TE,),
        in_specs=[
            pl.BlockSpec((TE, 16), lambda i: (i, 0)),
            pl.BlockSpec((TE, 16), lambda i: (i, 0)),
            pl.BlockSpec((8, 64), lambda i: (0, 0)),
            pl.BlockSpec((1, 64), lambda i: (0, 0)),
            pl.BlockSpec((64, 128), lambda i: (0, 0)),
            pl.BlockSpec((1, 128), lambda i: (0, 0)),
        ],
        out_specs=pl.BlockSpec((TE, 128), lambda i: (i, 0)),
        out_shape=jax.ShapeDtypeStruct((e, 128), jnp.float32),
    )(gs, gt, w1t, b1, w2t, b2)


def kernel(x, location_info, W1, b1, W2, b2, k):
    del x, k
    li = location_info
    pos = li[:, 6:8]
    idx = _knn_topk(pos)
    src = idx.reshape(-1)
    tgt = jnp.repeat(jnp.arange(N, dtype=jnp.int32), K)
    edge_index = jnp.stack([src, tgt], axis=0)

    li16 = jnp.pad(li, ((0, 0), (0, 6)))
    gs = _sc_gather(src, li16)
    gt = jnp.repeat(li16, K, axis=0)

    w1t = jnp.pad(jnp.transpose(W1), ((0, 4), (0, 0)))
    w2t = jnp.transpose(W2)
    edge_attr = _edge_mlp(gs, gt, w1t, b1.reshape(1, -1), w2t, b2.reshape(1, -1))
    return edge_index, edge_attr


# speedup vs baseline: 1.3469x; 1.0183x over previous
import functools

import jax
import jax.numpy as jnp
from jax import lax
from jax.experimental import pallas as pl
from jax.experimental.pallas import tpu as pltpu
from jax.experimental.pallas import tpu_sc as plsc

N = 10000
K = 32
NP = 10240
R = 200
C = 2048
TT = 200
TE = TT * K

_INF = 1e30
_BIGI = 2**30


def _bf16_rne(v):
    b = jax.lax.bitcast_convert_type(v, jnp.int32)
    r = (b + 0x7FFF + ((b >> 16) & 1)) & jnp.int32(-65536)
    return jax.lax.bitcast_convert_type(r, jnp.float32)


_M = 6


def _topk_body(rows_ref, cols_ref, idx_ref, xb2_scr, yb2_scr, sqc_scr):
    i = pl.program_id(0)
    xc_all = cols_ref[0:1, :]
    yc_all = cols_ref[1:2, :]
    xb2_scr[...] = -2.0 * _bf16_rne(xc_all)
    yb2_scr[...] = -2.0 * _bf16_rne(yc_all)
    sqc_scr[...] = xc_all * xc_all + yc_all * yc_all

    xr = rows_ref[:, 0:1]
    yr = rows_ref[:, 1:2]
    xrb = _bf16_rne(xr)
    yrb = _bf16_rne(yr)
    sqr = xr * xr + yr * yr
    row_gid = i * R + jax.lax.broadcasted_iota(jnp.int32, (R, 1), 0)
    lane32 = jax.lax.broadcasted_iota(jnp.int32, (R, K), 1)
    lane128 = jax.lax.broadcasted_iota(jnp.int32, (R, 128), 1)

    def d2_slice(start, width, lane_iota):
        xb2 = xb2_scr[0:1, pl.ds(start, width)]
        yb2 = yb2_scr[0:1, pl.ds(start, width)]
        sqc = sqc_scr[0:1, pl.ds(start, width)]
        d2 = (sqr + sqc) + (xrb * xb2 + yrb * yb2)
        cid = start + lane_iota
        return jnp.where(cid == row_gid, jnp.float32(1e10), d2), cid

    def extract_topk(a, ai):
        def extract(t4, ec):
            a, nv, ni = ec
            for u in range(16):
                t = t4 * 16 + u
                m = jnp.min(a, axis=1, keepdims=True)
                ci = jnp.min(jnp.where(a <= m, ai, _BIGI), axis=1, keepdims=True)
                a = jnp.where(ai == ci, _INF, a)
                nv = jnp.where(lane32 == t, m, nv)
                ni = jnp.where(lane32 == t, ci, ni)
            return a, nv, ni
        vals0 = jnp.full((R, K), _INF, jnp.float32)
        idxs0 = jnp.full((R, K), _BIGI, jnp.int32)
        _, nv, ni = jax.lax.fori_loop(0, K // 16, extract, (a, vals0, idxs0))
        return nv, ni

    def subtile_group(g, st):
        for u in range(20):
            s = g * 20 + u
            v = st[:_M]
            j = st[_M:]
            d2, cid = d2_slice(s * 128, 128, lane128)
            lt = [d2 < vm for vm in v]
            nv = [jnp.where(lt[0], d2, v[0])]
            nj = [jnp.where(lt[0], cid, j[0])]
            for m in range(1, _M):
                nv.append(jnp.where(lt[m - 1], v[m - 1], jnp.where(lt[m], d2, v[m])))
                nj.append(jnp.where(lt[m - 1], j[m - 1], jnp.where(lt[m], cid, j[m])))
            st = tuple(nv) + tuple(nj)
        return st

    init = (tuple(jnp.full((R, 128), _INF, jnp.float32) for _ in range(_M))
            + tuple(jnp.full((R, 128), _BIGI, jnp.int32) for _ in range(_M)))
    st = jax.lax.fori_loop(0, NP // 128 // 20, subtile_group, init)
    cand_v = jnp.concatenate(st[:_M - 1], axis=1)
    cand_i = jnp.concatenate(st[_M:2 * _M - 1], axis=1)
    nv, ni = extract_topk(cand_v, cand_i)
    tau = jnp.min(jnp.where(lane32 == K - 1, nv, _INF), axis=1, keepdims=True)
    unsafe = jnp.any(st[_M - 1] <= tau)

    def exact_path():
        laneC = jax.lax.broadcasted_iota(jnp.int32, (R, C), 1)

        def col_tile(c, carry):
            vals, idxs = carry
            d2, cid = d2_slice(c * C, C, laneC)
            a = jnp.concatenate([vals, d2], axis=1)
            ai = jnp.concatenate([idxs, cid], axis=1)
            def extract(t, ec):
                a, nv, ni = ec
                m = jnp.min(a, axis=1, keepdims=True)
                ci = jnp.min(jnp.where(a <= m, ai, _BIGI), axis=1, keepdims=True)
                a = jnp.where(ai == ci, _INF, a)
                nv = jnp.where(lane32 == t, m, nv)
                ni = jnp.where(lane32 == t, ci, ni)
                return a, nv, ni
            _, nv2, ni2 = jax.lax.fori_loop(0, K, extract, (a, vals, idxs))
            return nv2, ni2

        vals0 = jnp.full((R, K), _INF, jnp.float32)
        idxs0 = jnp.full((R, K), _BIGI, jnp.int32)
        _, idxs = jax.lax.fori_loop(0, NP // C, col_tile, (vals0, idxs0))
        return idxs

    idx_ref[...] = jax.lax.cond(unsafe, exact_path, lambda: ni)


def _mlp_body(gs_ref, gt_ref, w1t_ref, b1_ref, w2t_ref, b2_ref, out_ref):
    s4 = gs_ref[:, 4:5]
    s5 = gs_ref[:, 5:6]
    s6 = gs_ref[:, 6:7]
    s7 = gs_ref[:, 7:8]
    s8 = gs_ref[:, 8:9]
    s9 = gs_ref[:, 9:10]
    t4 = gt_ref[:, 4:5]
    t5 = gt_ref[:, 5:6]
    t6 = gt_ref[:, 6:7]
    t7 = gt_ref[:, 7:8]
    f1 = (s6 - t6) / s8
    f2 = (s7 - t7) / s9
    f3 = jnp.log(s4 / t4)
    f4 = jnp.log(s5 / t5)
    h = (f1 * w1t_ref[0:1, :] + f2 * w1t_ref[1:2, :]
         + f3 * w1t_ref[2:3, :] + f4 * w1t_ref[3:4, :]) + b1_ref[0:1, :]
    h = jnp.maximum(h, 0.0)
    o = jnp.dot(h, w2t_ref[...], preferred_element_type=jnp.float32)
    out_ref[...] = jnp.maximum(o + b2_ref[0:1, :], 0.0)


_E = N * K
_NW = 32
_BW = _E // _NW
_CHUNK = 2000
_NCH = _BW // _CHUNK


def _sc_gather(src, table):
    mesh = plsc.VectorSubcoreMesh(core_axis_name="c", subcore_axis_name="s")

    @functools.partial(
        pl.kernel, mesh=mesh,
        compiler_params=pltpu.CompilerParams(use_tc_tiling_on_sc=False),
        out_type=jax.ShapeDtypeStruct((_E, 16), jnp.float32),
        scratch_types=[
            pltpu.VMEM((_CHUNK,), jnp.int32),
            pltpu.VMEM((_CHUNK,), jnp.int32),
            pltpu.VMEM((_CHUNK, 16), jnp.float32),
            pltpu.VMEM((_CHUNK, 16), jnp.float32),
            pltpu.SemaphoreType.DMA,
            pltpu.SemaphoreType.DMA,
            pltpu.SemaphoreType.DMA,
            pltpu.SemaphoreType.DMA,
            pltpu.SemaphoreType.DMA,
            pltpu.SemaphoreType.DMA,
        ],
    )
    def gather_k(src_hbm, table_hbm, gs_hbm, idx_v0, idx_v1, rows_v0, rows_v1,
                 si0, si1, sg0, sg1, so0, so1):
        wid = lax.axis_index("s") * 2 + lax.axis_index("c")
        base = wid * _BW
        idx_v = (idx_v0, idx_v1)
        rows_v = (rows_v0, rows_v1)
        si = (si0, si1)
        sg = (sg0, sg1)
        so = (so0, so1)
        idx_pend = [None] * 2
        out_pend = [None] * 2
        idx_pend[0] = pltpu.async_copy(
            src_hbm.at[pl.ds(base, _CHUNK)], idx_v[0], si[0])
        for j in range(_NCH):
            b = j % 2
            nb = 1 - b
            if j + 1 < _NCH:
                idx_pend[nb] = pltpu.async_copy(
                    src_hbm.at[pl.ds(base + (j + 1) * _CHUNK, _CHUNK)],
                    idx_v[nb], si[nb])
            idx_pend[b].wait()
            if out_pend[b] is not None:
                out_pend[b].wait()
            pltpu.async_copy(table_hbm.at[idx_v[b]], rows_v[b], sg[b]).wait()
            out_pend[b] = pltpu.async_copy(
                rows_v[b], gs_hbm.at[pl.ds(base + j * _CHUNK, _CHUNK)], so[b])
        for p in out_pend:
            if p is not None:
                p.wait()

    return gather_k(src, table)


def _knn_topk(pos):
    pos_t = jnp.transpose(pos)
    pad = jnp.full((2, NP - N), 1e4, jnp.float32)
    cols = jnp.concatenate([pos_t, pad], axis=1)
    return pl.pallas_call(
        _topk_body,
        grid=(N // R,),
        in_specs=[
            pl.BlockSpec((R, 2), lambda i: (i, 0)),
            pl.BlockSpec((2, NP), lambda i: (0, 0)),
        ],
        out_specs=pl.BlockSpec((R, K), lambda i: (i, 0)),
        out_shape=jax.ShapeDtypeStruct((N, K), jnp.int32),
        scratch_shapes=[
            pltpu.VMEM((1, NP), jnp.float32),
            pltpu.VMEM((1, NP), jnp.float32),
            pltpu.VMEM((1, NP), jnp.float32),
        ],
    )(pos, cols)


def _edge_mlp(gs, gt, w1t, b1, w2t, b2):
    e = gs.shape[0]
    return pl.pallas_call(
        _mlp_body,
        grid=(e // TE,),
        in_specs=[
            pl.BlockSpec((TE, 16), lambda i: (i, 0)),
            pl.BlockSpec((TE, 16), lambda i: (i, 0)),
            pl.BlockSpec((8, 64), lambda i: (0, 0)),
            pl.BlockSpec((1, 64), lambda i: (0, 0)),
            pl.BlockSpec((64, 128), lambda i: (0, 0)),
            pl.BlockSpec((1, 128), lambda i: (0, 0)),
        ],
        out_specs=pl.BlockSpec((TE, 128), lambda i: (i, 0)),
        out_shape=jax.ShapeDtypeStruct((e, 128), jnp.float32),
    )(gs, gt, w1t, b1, w2t, b2)


def kernel(x, location_info, W1, b1, W2, b2, k):
    del x, k
    li = location_info
    pos = li[:, 6:8]
    idx = _knn_topk(pos)
    src = idx.reshape(-1)
    tgt = jnp.repeat(jnp.arange(N, dtype=jnp.int32), K)
    edge_index = jnp.stack([src, tgt], axis=0)

    li16 = jnp.pad(li, ((0, 0), (0, 6)))
    gs = _sc_gather(src, li16)
    gt = jnp.repeat(li16, K, axis=0)

    w1t = jnp.pad(jnp.transpose(W1), ((0, 4), (0, 0)))
    w2t = jnp.transpose(W2)
    edge_attr = _edge_mlp(gs, gt, w1t, b1.reshape(1, -1), w2t, b2.reshape(1, -1))
    return edge_index, edge_attr


# speedup vs baseline: 1.3836x; 1.0272x over previous
import functools

import jax
import jax.numpy as jnp
from jax import lax
from jax.experimental import pallas as pl
from jax.experimental.pallas import tpu as pltpu
from jax.experimental.pallas import tpu_sc as plsc

N = 10000
K = 32
NP = 10240
R = 200
C = 2048
TT = 200
TE = TT * K

_INF = 1e30
_BIGI = 2**30


def _bf16_rne(v):
    b = jax.lax.bitcast_convert_type(v, jnp.int32)
    r = (b + 0x7FFF + ((b >> 16) & 1)) & jnp.int32(-65536)
    return jax.lax.bitcast_convert_type(r, jnp.float32)


_M = 6


def _topk_body(rows_ref, cols_ref, idx_ref, xb2_scr, yb2_scr, sqc_scr):
    i = pl.program_id(0)
    xc_all = cols_ref[0:1, :]
    yc_all = cols_ref[1:2, :]
    xb2_scr[...] = -2.0 * _bf16_rne(xc_all)
    yb2_scr[...] = -2.0 * _bf16_rne(yc_all)
    sqc_scr[...] = xc_all * xc_all + yc_all * yc_all

    xr = rows_ref[:, 0:1]
    yr = rows_ref[:, 1:2]
    xrb = _bf16_rne(xr)
    yrb = _bf16_rne(yr)
    sqr = xr * xr + yr * yr
    row_gid = i * R + jax.lax.broadcasted_iota(jnp.int32, (R, 1), 0)
    lane32 = jax.lax.broadcasted_iota(jnp.int32, (R, K), 1)
    lane128 = jax.lax.broadcasted_iota(jnp.int32, (R, 128), 1)

    def d2_slice(start, width, lane_iota):
        xb2 = xb2_scr[0:1, pl.ds(start, width)]
        yb2 = yb2_scr[0:1, pl.ds(start, width)]
        sqc = sqc_scr[0:1, pl.ds(start, width)]
        d2 = (sqr + sqc) + (xrb * xb2 + yrb * yb2)
        cid = start + lane_iota
        return jnp.where(cid == row_gid, jnp.float32(1e10), d2), cid

    def extract_topk(a, ai):
        def extract(t4, ec):
            a, nv, ni = ec
            for u in range(32):
                t = t4 * 32 + u
                m = jnp.min(a, axis=1, keepdims=True)
                ci = jnp.min(jnp.where(a <= m, ai, _BIGI), axis=1, keepdims=True)
                a = jnp.where(ai == ci, _INF, a)
                nv = jnp.where(lane32 == t, m, nv)
                ni = jnp.where(lane32 == t, ci, ni)
            return a, nv, ni
        vals0 = jnp.full((R, K), _INF, jnp.float32)
        idxs0 = jnp.full((R, K), _BIGI, jnp.int32)
        _, nv, ni = jax.lax.fori_loop(0, K // 32, extract, (a, vals0, idxs0))
        return nv, ni

    def subtile_group(g, st):
        for u in range(40):
            s = g * 40 + u
            v = st[:_M]
            j = st[_M:]
            d2, cid = d2_slice(s * 128, 128, lane128)
            lt = [d2 < vm for vm in v]
            nv = [jnp.where(lt[0], d2, v[0])]
            nj = [jnp.where(lt[0], cid, j[0])]
            for m in range(1, _M):
                nv.append(jnp.where(lt[m - 1], v[m - 1], jnp.where(lt[m], d2, v[m])))
                nj.append(jnp.where(lt[m - 1], j[m - 1], jnp.where(lt[m], cid, j[m])))
            st = tuple(nv) + tuple(nj)
        return st

    init = (tuple(jnp.full((R, 128), _INF, jnp.float32) for _ in range(_M))
            + tuple(jnp.full((R, 128), _BIGI, jnp.int32) for _ in range(_M)))
    st = jax.lax.fori_loop(0, NP // 128 // 40, subtile_group, init)
    cand_v = jnp.concatenate(st[:_M - 1], axis=1)
    cand_i = jnp.concatenate(st[_M:2 * _M - 1], axis=1)
    nv, ni = extract_topk(cand_v, cand_i)
    tau = jnp.min(jnp.where(lane32 == K - 1, nv, _INF), axis=1, keepdims=True)
    unsafe = jnp.any(st[_M - 1] <= tau)

    def exact_path():
        laneC = jax.lax.broadcasted_iota(jnp.int32, (R, C), 1)

        def col_tile(c, carry):
            vals, idxs = carry
            d2, cid = d2_slice(c * C, C, laneC)
            a = jnp.concatenate([vals, d2], axis=1)
            ai = jnp.concatenate([idxs, cid], axis=1)
            def extract(t, ec):
                a, nv, ni = ec
                m = jnp.min(a, axis=1, keepdims=True)
                ci = jnp.min(jnp.where(a <= m, ai, _BIGI), axis=1, keepdims=True)
                a = jnp.where(ai == ci, _INF, a)
                nv = jnp.where(lane32 == t, m, nv)
                ni = jnp.where(lane32 == t, ci, ni)
                return a, nv, ni
            _, nv2, ni2 = jax.lax.fori_loop(0, K, extract, (a, vals, idxs))
            return nv2, ni2

        vals0 = jnp.full((R, K), _INF, jnp.float32)
        idxs0 = jnp.full((R, K), _BIGI, jnp.int32)
        _, idxs = jax.lax.fori_loop(0, NP // C, col_tile, (vals0, idxs0))
        return idxs

    idx_ref[...] = jax.lax.cond(unsafe, exact_path, lambda: ni)


def _mlp_body(gs_ref, gt_ref, w1t_ref, b1_ref, w2t_ref, b2_ref, out_ref):
    s4 = gs_ref[:, 4:5]
    s5 = gs_ref[:, 5:6]
    s6 = gs_ref[:, 6:7]
    s7 = gs_ref[:, 7:8]
    s8 = gs_ref[:, 8:9]
    s9 = gs_ref[:, 9:10]
    t4 = gt_ref[:, 4:5]
    t5 = gt_ref[:, 5:6]
    t6 = gt_ref[:, 6:7]
    t7 = gt_ref[:, 7:8]
    f1 = (s6 - t6) / s8
    f2 = (s7 - t7) / s9
    f3 = jnp.log(s4 / t4)
    f4 = jnp.log(s5 / t5)
    h = (f1 * w1t_ref[0:1, :] + f2 * w1t_ref[1:2, :]
         + f3 * w1t_ref[2:3, :] + f4 * w1t_ref[3:4, :]) + b1_ref[0:1, :]
    h = jnp.maximum(h, 0.0)
    o = jnp.dot(h, w2t_ref[...], preferred_element_type=jnp.float32)
    out_ref[...] = jnp.maximum(o + b2_ref[0:1, :], 0.0)


_E = N * K
_NW = 32
_BW = _E // _NW
_CHUNK = 2000
_NCH = _BW // _CHUNK


def _sc_gather(src, table):
    mesh = plsc.VectorSubcoreMesh(core_axis_name="c", subcore_axis_name="s")

    @functools.partial(
        pl.kernel, mesh=mesh,
        compiler_params=pltpu.CompilerParams(use_tc_tiling_on_sc=False),
        out_type=jax.ShapeDtypeStruct((_E, 16), jnp.float32),
        scratch_types=[
            pltpu.VMEM((_CHUNK,), jnp.int32),
            pltpu.VMEM((_CHUNK,), jnp.int32),
            pltpu.VMEM((_CHUNK, 16), jnp.float32),
            pltpu.VMEM((_CHUNK, 16), jnp.float32),
            pltpu.SemaphoreType.DMA,
            pltpu.SemaphoreType.DMA,
            pltpu.SemaphoreType.DMA,
            pltpu.SemaphoreType.DMA,
            pltpu.SemaphoreType.DMA,
            pltpu.SemaphoreType.DMA,
        ],
    )
    def gather_k(src_hbm, table_hbm, gs_hbm, idx_v0, idx_v1, rows_v0, rows_v1,
                 si0, si1, sg0, sg1, so0, so1):
        wid = lax.axis_index("s") * 2 + lax.axis_index("c")
        base = wid * _BW
        idx_v = (idx_v0, idx_v1)
        rows_v = (rows_v0, rows_v1)
        si = (si0, si1)
        sg = (sg0, sg1)
        so = (so0, so1)
        idx_pend = [None] * 2
        out_pend = [None] * 2
        idx_pend[0] = pltpu.async_copy(
            src_hbm.at[pl.ds(base, _CHUNK)], idx_v[0], si[0])
        for j in range(_NCH):
            b = j % 2
            nb = 1 - b
            if j + 1 < _NCH:
                idx_pend[nb] = pltpu.async_copy(
                    src_hbm.at[pl.ds(base + (j + 1) * _CHUNK, _CHUNK)],
                    idx_v[nb], si[nb])
            idx_pend[b].wait()
            if out_pend[b] is not None:
                out_pend[b].wait()
            pltpu.async_copy(table_hbm.at[idx_v[b]], rows_v[b], sg[b]).wait()
            out_pend[b] = pltpu.async_copy(
                rows_v[b], gs_hbm.at[pl.ds(base + j * _CHUNK, _CHUNK)], so[b])
        for p in out_pend:
            if p is not None:
                p.wait()

    return gather_k(src, table)


def _knn_topk(pos):
    pos_t = jnp.transpose(pos)
    pad = jnp.full((2, NP - N), 1e4, jnp.float32)
    cols = jnp.concatenate([pos_t, pad], axis=1)
    return pl.pallas_call(
        _topk_body,
        grid=(N // R,),
        in_specs=[
            pl.BlockSpec((R, 2), lambda i: (i, 0)),
            pl.BlockSpec((2, NP), lambda i: (0, 0)),
        ],
        out_specs=pl.BlockSpec((R, K), lambda i: (i, 0)),
        out_shape=jax.ShapeDtypeStruct((N, K), jnp.int32),
        scratch_shapes=[
            pltpu.VMEM((1, NP), jnp.float32),
            pltpu.VMEM((1, NP), jnp.float32),
            pltpu.VMEM((1, NP), jnp.float32),
        ],
    )(pos, cols)


def _edge_mlp(gs, gt, w1t, b1, w2t, b2):
    e = gs.shape[0]
    return pl.pallas_call(
        _mlp_body,
        grid=(e // TE,),
        in_specs=[
            pl.BlockSpec((TE, 16), lambda i: (i, 0)),
            pl.BlockSpec((TE, 16), lambda i: (i, 0)),
            pl.BlockSpec((8, 64), lambda i: (0, 0)),
            pl.BlockSpec((1, 64), lambda i: (0, 0)),
            pl.BlockSpec((64, 128), lambda i: (0, 0)),
            pl.BlockSpec((1, 128), lambda i: (0, 0)),
        ],
        out_specs=pl.BlockSpec((TE, 128), lambda i: (i, 0)),
        out_shape=jax.ShapeDtypeStruct((e, 128), jnp.float32),
    )(gs, gt, w1t, b1, w2t, b2)


def kernel(x, location_info, W1, b1, W2, b2, k):
    del x, k
    li = location_info
    pos = li[:, 6:8]
    idx = _knn_topk(pos)
    src = idx.reshape(-1)
    tgt = jnp.repeat(jnp.arange(N, dtype=jnp.int32), K)
    edge_index = jnp.stack([src, tgt], axis=0)

    li16 = jnp.pad(li, ((0, 0), (0, 6)))
    gs = _sc_gather(src, li16)
    gt = jnp.repeat(li16, K, axis=0)

    w1t = jnp.pad(jnp.transpose(W1), ((0, 4), (0, 0)))
    w2t = jnp.transpose(W2)
    edge_attr = _edge_mlp(gs, gt, w1t, b1.reshape(1, -1), w2t, b2.reshape(1, -1))
    return edge_index, edge_attr


# speedup vs baseline: 1.7576x; 1.2703x over previous
import functools

import jax
import jax.numpy as jnp
from jax import lax
from jax.experimental import pallas as pl
from jax.experimental.pallas import tpu as pltpu
from jax.experimental.pallas import tpu_sc as plsc

N = 10000
K = 32
NP = 10240
R = 200
C = 2048
TT = 200
TE = TT * K

_INF = 1e30
_BIGI = 2**30


def _bf16_rne(v):
    b = jax.lax.bitcast_convert_type(v, jnp.int32)
    r = (b + 0x7FFF + ((b >> 16) & 1)) & jnp.int32(-65536)
    return jax.lax.bitcast_convert_type(r, jnp.float32)


_M = 6


def _topk_body(rows_ref, cols_ref, idx_ref, xb2_scr, yb2_scr, sqc_scr):
    i = pl.program_id(0)
    xc_all = cols_ref[0:1, :]
    yc_all = cols_ref[1:2, :]
    xb2_scr[...] = -2.0 * _bf16_rne(xc_all)
    yb2_scr[...] = -2.0 * _bf16_rne(yc_all)
    sqc_scr[...] = xc_all * xc_all + yc_all * yc_all

    xr = rows_ref[:, 0:1]
    yr = rows_ref[:, 1:2]
    xrb = _bf16_rne(xr)
    yrb = _bf16_rne(yr)
    sqr = xr * xr + yr * yr
    row_gid = i * R + jax.lax.broadcasted_iota(jnp.int32, (R, 1), 0)
    lane32 = jax.lax.broadcasted_iota(jnp.int32, (R, K), 1)
    lane128 = jax.lax.broadcasted_iota(jnp.int32, (R, 128), 1)

    def d2_slice(start, width, lane_iota):
        xb2 = xb2_scr[0:1, pl.ds(start, width)]
        yb2 = yb2_scr[0:1, pl.ds(start, width)]
        sqc = sqc_scr[0:1, pl.ds(start, width)]
        d2 = (sqr + sqc) + (xrb * xb2 + yrb * yb2)
        cid = start + lane_iota
        return jnp.where(cid == row_gid, jnp.float32(1e10), d2), cid

    def extract_topk(a, ai):
        def extract(t4, ec):
            a, nv, ni = ec
            for u in range(32):
                t = t4 * 32 + u
                m = jnp.min(a, axis=1, keepdims=True)
                ci = jnp.min(jnp.where(a <= m, ai, _BIGI), axis=1, keepdims=True)
                a = jnp.where(ai == ci, _INF, a)
                nv = jnp.where(lane32 == t, m, nv)
                ni = jnp.where(lane32 == t, ci, ni)
            return a, nv, ni
        vals0 = jnp.full((R, K), _INF, jnp.float32)
        idxs0 = jnp.full((R, K), _BIGI, jnp.int32)
        _, nv, ni = jax.lax.fori_loop(0, K // 32, extract, (a, vals0, idxs0))
        return nv, ni

    def subtile_group(g, st):
        for u in range(80):
            s = g * 80 + u
            v = st[:_M]
            j = st[_M:]
            d2, cid = d2_slice(s * 128, 128, lane128)
            lt = [d2 < vm for vm in v]
            nv = [jnp.where(lt[0], d2, v[0])]
            nj = [jnp.where(lt[0], cid, j[0])]
            for m in range(1, _M):
                nv.append(jnp.where(lt[m - 1], v[m - 1], jnp.where(lt[m], d2, v[m])))
                nj.append(jnp.where(lt[m - 1], j[m - 1], jnp.where(lt[m], cid, j[m])))
            st = tuple(nv) + tuple(nj)
        return st

    init = (tuple(jnp.full((R, 128), _INF, jnp.float32) for _ in range(_M))
            + tuple(jnp.full((R, 128), _BIGI, jnp.int32) for _ in range(_M)))
    st = jax.lax.fori_loop(0, NP // 128 // 80, subtile_group, init)
    cand_v = jnp.concatenate(st[:_M - 1], axis=1)
    cand_i = jnp.concatenate(st[_M:2 * _M - 1], axis=1)
    nv, ni = extract_topk(cand_v, cand_i)
    tau = jnp.min(jnp.where(lane32 == K - 1, nv, _INF), axis=1, keepdims=True)
    unsafe = jnp.any(st[_M - 1] <= tau)

    def exact_path():
        laneC = jax.lax.broadcasted_iota(jnp.int32, (R, C), 1)

        def col_tile(c, carry):
            vals, idxs = carry
            d2, cid = d2_slice(c * C, C, laneC)
            a = jnp.concatenate([vals, d2], axis=1)
            ai = jnp.concatenate([idxs, cid], axis=1)
            def extract(t, ec):
                a, nv, ni = ec
                m = jnp.min(a, axis=1, keepdims=True)
                ci = jnp.min(jnp.where(a <= m, ai, _BIGI), axis=1, keepdims=True)
                a = jnp.where(ai == ci, _INF, a)
                nv = jnp.where(lane32 == t, m, nv)
                ni = jnp.where(lane32 == t, ci, ni)
                return a, nv, ni
            _, nv2, ni2 = jax.lax.fori_loop(0, K, extract, (a, vals, idxs))
            return nv2, ni2

        vals0 = jnp.full((R, K), _INF, jnp.float32)
        idxs0 = jnp.full((R, K), _BIGI, jnp.int32)
        _, idxs = jax.lax.fori_loop(0, NP // C, col_tile, (vals0, idxs0))
        return idxs

    idx_ref[...] = jax.lax.cond(unsafe, exact_path, lambda: ni)


def _mlp_body(gs_ref, gt_ref, w1t_ref, b1_ref, w2t_ref, b2_ref, out_ref):
    s4 = gs_ref[:, 4:5]
    s5 = gs_ref[:, 5:6]
    s6 = gs_ref[:, 6:7]
    s7 = gs_ref[:, 7:8]
    s8 = gs_ref[:, 8:9]
    s9 = gs_ref[:, 9:10]
    t4 = gt_ref[:, 4:5]
    t5 = gt_ref[:, 5:6]
    t6 = gt_ref[:, 6:7]
    t7 = gt_ref[:, 7:8]
    f1 = (s6 - t6) / s8
    f2 = (s7 - t7) / s9
    f3 = jnp.log(s4 / t4)
    f4 = jnp.log(s5 / t5)
    h = (f1 * w1t_ref[0:1, :] + f2 * w1t_ref[1:2, :]
         + f3 * w1t_ref[2:3, :] + f4 * w1t_ref[3:4, :]) + b1_ref[0:1, :]
    h = jnp.maximum(h, 0.0)
    o = jnp.dot(h, w2t_ref[...], preferred_element_type=jnp.float32)
    out_ref[...] = jnp.maximum(o + b2_ref[0:1, :], 0.0)


_E = N * K
_NW = 32
_BW = _E // _NW
_CHUNK = 2000
_NCH = _BW // _CHUNK


def _sc_gather(src, table):
    mesh = plsc.VectorSubcoreMesh(core_axis_name="c", subcore_axis_name="s")

    @functools.partial(
        pl.kernel, mesh=mesh,
        compiler_params=pltpu.CompilerParams(use_tc_tiling_on_sc=False),
        out_type=jax.ShapeDtypeStruct((_E, 16), jnp.float32),
        scratch_types=[
            pltpu.VMEM((_CHUNK,), jnp.int32),
            pltpu.VMEM((_CHUNK,), jnp.int32),
            pltpu.VMEM((_CHUNK, 16), jnp.float32),
            pltpu.VMEM((_CHUNK, 16), jnp.float32),
            pltpu.SemaphoreType.DMA,
            pltpu.SemaphoreType.DMA,
            pltpu.SemaphoreType.DMA,
            pltpu.SemaphoreType.DMA,
            pltpu.SemaphoreType.DMA,
            pltpu.SemaphoreType.DMA,
        ],
    )
    def gather_k(src_hbm, table_hbm, gs_hbm, idx_v0, idx_v1, rows_v0, rows_v1,
                 si0, si1, sg0, sg1, so0, so1):
        wid = lax.axis_index("s") * 2 + lax.axis_index("c")
        base = wid * _BW
        idx_v = (idx_v0, idx_v1)
        rows_v = (rows_v0, rows_v1)
        si = (si0, si1)
        sg = (sg0, sg1)
        so = (so0, so1)
        idx_pend = [None] * 2
        out_pend = [None] * 2
        idx_pend[0] = pltpu.async_copy(
            src_hbm.at[pl.ds(base, _CHUNK)], idx_v[0], si[0])
        for j in range(_NCH):
            b = j % 2
            nb = 1 - b
            if j + 1 < _NCH:
                idx_pend[nb] = pltpu.async_copy(
                    src_hbm.at[pl.ds(base + (j + 1) * _CHUNK, _CHUNK)],
                    idx_v[nb], si[nb])
            idx_pend[b].wait()
            if out_pend[b] is not None:
                out_pend[b].wait()
            pltpu.async_copy(table_hbm.at[idx_v[b]], rows_v[b], sg[b]).wait()
            out_pend[b] = pltpu.async_copy(
                rows_v[b], gs_hbm.at[pl.ds(base + j * _CHUNK, _CHUNK)], so[b])
        for p in out_pend:
            if p is not None:
                p.wait()

    return gather_k(src, table)


def _knn_topk(pos):
    pos_t = jnp.transpose(pos)
    pad = jnp.full((2, NP - N), 1e4, jnp.float32)
    cols = jnp.concatenate([pos_t, pad], axis=1)
    return pl.pallas_call(
        _topk_body,
        grid=(N // R,),
        in_specs=[
            pl.BlockSpec((R, 2), lambda i: (i, 0)),
            pl.BlockSpec((2, NP), lambda i: (0, 0)),
        ],
        out_specs=pl.BlockSpec((R, K), lambda i: (i, 0)),
        out_shape=jax.ShapeDtypeStruct((N, K), jnp.int32),
        scratch_shapes=[
            pltpu.VMEM((1, NP), jnp.float32),
            pltpu.VMEM((1, NP), jnp.float32),
            pltpu.VMEM((1, NP), jnp.float32),
        ],
    )(pos, cols)


def _edge_mlp(gs, gt, w1t, b1, w2t, b2):
    e = gs.shape[0]
    return pl.pallas_call(
        _mlp_body,
        grid=(e // TE,),
        in_specs=[
            pl.BlockSpec((TE, 16), lambda i: (i, 0)),
            pl.BlockSpec((TE, 16), lambda i: (i, 0)),
            pl.BlockSpec((8, 64), lambda i: (0, 0)),
            pl.BlockSpec((1, 64), lambda i: (0, 0)),
            pl.BlockSpec((64, 128), lambda i: (0, 0)),
            pl.BlockSpec((1, 128), lambda i: (0, 0)),
        ],
        out_specs=pl.BlockSpec((TE, 128), lambda i: (i, 0)),
        out_shape=jax.ShapeDtypeStruct((e, 128), jnp.float32),
    )(gs, gt, w1t, b1, w2t, b2)


def kernel(x, location_info, W1, b1, W2, b2, k):
    del x, k
    li = location_info
    pos = li[:, 6:8]
    idx = _knn_topk(pos)
    src = idx.reshape(-1)
    tgt = jnp.repeat(jnp.arange(N, dtype=jnp.int32), K)
    edge_index = jnp.stack([src, tgt], axis=0)

    li16 = jnp.pad(li, ((0, 0), (0, 6)))
    gs = _sc_gather(src, li16)
    gt = jnp.repeat(li16, K, axis=0)

    w1t = jnp.pad(jnp.transpose(W1), ((0, 4), (0, 0)))
    w2t = jnp.transpose(W2)
    edge_attr = _edge_mlp(gs, gt, w1t, b1.reshape(1, -1), w2t, b2.reshape(1, -1))
    return edge_index, edge_attr


# speedup vs baseline: 1.7595x; 1.0011x over previous
import functools

import jax
import jax.numpy as jnp
from jax import lax
from jax.experimental import pallas as pl
from jax.experimental.pallas import tpu as pltpu
from jax.experimental.pallas import tpu_sc as plsc

N = 10000
K = 32
NP = 10240
R = 200
C = 2048
TT = 200
TE = TT * K

_INF = 1e30
_BIGI = 2**30


def _bf16_rne(v):
    b = jax.lax.bitcast_convert_type(v, jnp.int32)
    r = (b + 0x7FFF + ((b >> 16) & 1)) & jnp.int32(-65536)
    return jax.lax.bitcast_convert_type(r, jnp.float32)


_M = 6


def _topk_body(rows_ref, cols_ref, idx_ref, xb2_scr, yb2_scr, sqc_scr):
    i = pl.program_id(0)
    xc_all = cols_ref[0:1, :]
    yc_all = cols_ref[1:2, :]
    xb2_scr[...] = -2.0 * _bf16_rne(xc_all)
    yb2_scr[...] = -2.0 * _bf16_rne(yc_all)
    sqc_scr[...] = xc_all * xc_all + yc_all * yc_all

    xr = rows_ref[:, 0:1]
    yr = rows_ref[:, 1:2]
    xrb = _bf16_rne(xr)
    yrb = _bf16_rne(yr)
    sqr = xr * xr + yr * yr
    row_gid = i * R + jax.lax.broadcasted_iota(jnp.int32, (R, 1), 0)
    lane32 = jax.lax.broadcasted_iota(jnp.int32, (R, K), 1)
    lane128 = jax.lax.broadcasted_iota(jnp.int32, (R, 128), 1)

    def d2_slice(start, width, lane_iota):
        xb2 = xb2_scr[0:1, pl.ds(start, width)]
        yb2 = yb2_scr[0:1, pl.ds(start, width)]
        sqc = sqc_scr[0:1, pl.ds(start, width)]
        d2 = (sqr + sqc) + (xrb * xb2 + yrb * yb2)
        cid = start + lane_iota
        return jnp.where(cid == row_gid, jnp.float32(1e10), d2), cid

    def extract_topk(a, ai):
        def extract(t4, ec):
            a, nv, ni = ec
            for u in range(32):
                t = t4 * 32 + u
                m = jnp.min(a, axis=1, keepdims=True)
                ci = jnp.min(jnp.where(a <= m, ai, _BIGI), axis=1, keepdims=True)
                a = jnp.where(ai == ci, _INF, a)
                nv = jnp.where(lane32 == t, m, nv)
                ni = jnp.where(lane32 == t, ci, ni)
            return a, nv, ni
        vals0 = jnp.full((R, K), _INF, jnp.float32)
        idxs0 = jnp.full((R, K), _BIGI, jnp.int32)
        _, nv, ni = extract(0, (a, vals0, idxs0))
        return nv, ni

    def subtile_group(g, st):
        for u in range(80):
            s = g * 80 + u
            v = st[:_M]
            j = st[_M:]
            d2, cid = d2_slice(s * 128, 128, lane128)
            lt = [d2 < vm for vm in v]
            nv = [jnp.where(lt[0], d2, v[0])]
            nj = [jnp.where(lt[0], cid, j[0])]
            for m in range(1, _M):
                nv.append(jnp.where(lt[m - 1], v[m - 1], jnp.where(lt[m], d2, v[m])))
                nj.append(jnp.where(lt[m - 1], j[m - 1], jnp.where(lt[m], cid, j[m])))
            st = tuple(nv) + tuple(nj)
        return st

    init = (tuple(jnp.full((R, 128), _INF, jnp.float32) for _ in range(_M))
            + tuple(jnp.full((R, 128), _BIGI, jnp.int32) for _ in range(_M)))
    st = subtile_group(0, init)
    cand_v = jnp.concatenate(st[:_M - 1], axis=1)
    cand_i = jnp.concatenate(st[_M:2 * _M - 1], axis=1)
    nv, ni = extract_topk(cand_v, cand_i)
    tau = jnp.min(jnp.where(lane32 == K - 1, nv, _INF), axis=1, keepdims=True)
    unsafe = jnp.any(st[_M - 1] <= tau)

    def exact_path():
        laneC = jax.lax.broadcasted_iota(jnp.int32, (R, C), 1)

        def col_tile(c, carry):
            vals, idxs = carry
            d2, cid = d2_slice(c * C, C, laneC)
            a = jnp.concatenate([vals, d2], axis=1)
            ai = jnp.concatenate([idxs, cid], axis=1)
            def extract(t, ec):
                a, nv, ni = ec
                m = jnp.min(a, axis=1, keepdims=True)
                ci = jnp.min(jnp.where(a <= m, ai, _BIGI), axis=1, keepdims=True)
                a = jnp.where(ai == ci, _INF, a)
                nv = jnp.where(lane32 == t, m, nv)
                ni = jnp.where(lane32 == t, ci, ni)
                return a, nv, ni
            _, nv2, ni2 = jax.lax.fori_loop(0, K, extract, (a, vals, idxs))
            return nv2, ni2

        vals0 = jnp.full((R, K), _INF, jnp.float32)
        idxs0 = jnp.full((R, K), _BIGI, jnp.int32)
        _, idxs = jax.lax.fori_loop(0, NP // C, col_tile, (vals0, idxs0))
        return idxs

    idx_ref[...] = jax.lax.cond(unsafe, exact_path, lambda: ni)


def _mlp_body(gs_ref, gt_ref, w1t_ref, b1_ref, w2t_ref, b2_ref, out_ref):
    s4 = gs_ref[:, 4:5]
    s5 = gs_ref[:, 5:6]
    s6 = gs_ref[:, 6:7]
    s7 = gs_ref[:, 7:8]
    s8 = gs_ref[:, 8:9]
    s9 = gs_ref[:, 9:10]
    t4 = gt_ref[:, 4:5]
    t5 = gt_ref[:, 5:6]
    t6 = gt_ref[:, 6:7]
    t7 = gt_ref[:, 7:8]
    f1 = (s6 - t6) / s8
    f2 = (s7 - t7) / s9
    f3 = jnp.log(s4 / t4)
    f4 = jnp.log(s5 / t5)
    h = (f1 * w1t_ref[0:1, :] + f2 * w1t_ref[1:2, :]
         + f3 * w1t_ref[2:3, :] + f4 * w1t_ref[3:4, :]) + b1_ref[0:1, :]
    h = jnp.maximum(h, 0.0)
    o = jnp.dot(h, w2t_ref[...], preferred_element_type=jnp.float32)
    out_ref[...] = jnp.maximum(o + b2_ref[0:1, :], 0.0)


_E = N * K
_NW = 32
_BW = _E // _NW
_CHUNK = 2000
_NCH = _BW // _CHUNK


def _sc_gather(src, table):
    mesh = plsc.VectorSubcoreMesh(core_axis_name="c", subcore_axis_name="s")

    @functools.partial(
        pl.kernel, mesh=mesh,
        compiler_params=pltpu.CompilerParams(use_tc_tiling_on_sc=False),
        out_type=jax.ShapeDtypeStruct((_E, 16), jnp.float32),
        scratch_types=[
            pltpu.VMEM((_CHUNK,), jnp.int32),
            pltpu.VMEM((_CHUNK,), jnp.int32),
            pltpu.VMEM((_CHUNK, 16), jnp.float32),
            pltpu.VMEM((_CHUNK, 16), jnp.float32),
            pltpu.SemaphoreType.DMA,
            pltpu.SemaphoreType.DMA,
            pltpu.SemaphoreType.DMA,
            pltpu.SemaphoreType.DMA,
            pltpu.SemaphoreType.DMA,
            pltpu.SemaphoreType.DMA,
        ],
    )
    def gather_k(src_hbm, table_hbm, gs_hbm, idx_v0, idx_v1, rows_v0, rows_v1,
                 si0, si1, sg0, sg1, so0, so1):
        wid = lax.axis_index("s") * 2 + lax.axis_index("c")
        base = wid * _BW
        idx_v = (idx_v0, idx_v1)
        rows_v = (rows_v0, rows_v1)
        si = (si0, si1)
        sg = (sg0, sg1)
        so = (so0, so1)
        idx_pend = [None] * 2
        out_pend = [None] * 2
        idx_pend[0] = pltpu.async_copy(
            src_hbm.at[pl.ds(base, _CHUNK)], idx_v[0], si[0])
        for j in range(_NCH):
            b = j % 2
            nb = 1 - b
            if j + 1 < _NCH:
                idx_pend[nb] = pltpu.async_copy(
                    src_hbm.at[pl.ds(base + (j + 1) * _CHUNK, _CHUNK)],
                    idx_v[nb], si[nb])
            idx_pend[b].wait()
            if out_pend[b] is not None:
                out_pend[b].wait()
            pltpu.async_copy(table_hbm.at[idx_v[b]], rows_v[b], sg[b]).wait()
            out_pend[b] = pltpu.async_copy(
                rows_v[b], gs_hbm.at[pl.ds(base + j * _CHUNK, _CHUNK)], so[b])
        for p in out_pend:
            if p is not None:
                p.wait()

    return gather_k(src, table)


def _knn_topk(pos):
    pos_t = jnp.transpose(pos)
    pad = jnp.full((2, NP - N), 1e4, jnp.float32)
    cols = jnp.concatenate([pos_t, pad], axis=1)
    return pl.pallas_call(
        _topk_body,
        grid=(N // R,),
        in_specs=[
            pl.BlockSpec((R, 2), lambda i: (i, 0)),
            pl.BlockSpec((2, NP), lambda i: (0, 0)),
        ],
        out_specs=pl.BlockSpec((R, K), lambda i: (i, 0)),
        out_shape=jax.ShapeDtypeStruct((N, K), jnp.int32),
        scratch_shapes=[
            pltpu.VMEM((1, NP), jnp.float32),
            pltpu.VMEM((1, NP), jnp.float32),
            pltpu.VMEM((1, NP), jnp.float32),
        ],
    )(pos, cols)


def _edge_mlp(gs, gt, w1t, b1, w2t, b2):
    e = gs.shape[0]
    return pl.pallas_call(
        _mlp_body,
        grid=(e // TE,),
        in_specs=[
            pl.BlockSpec((TE, 16), lambda i: (i, 0)),
            pl.BlockSpec((TE, 16), lambda i: (i, 0)),
            pl.BlockSpec((8, 64), lambda i: (0, 0)),
            pl.BlockSpec((1, 64), lambda i: (0, 0)),
            pl.BlockSpec((64, 128), lambda i: (0, 0)),
            pl.BlockSpec((1, 128), lambda i: (0, 0)),
        ],
        out_specs=pl.BlockSpec((TE, 128), lambda i: (i, 0)),
        out_shape=jax.ShapeDtypeStruct((e, 128), jnp.float32),
    )(gs, gt, w1t, b1, w2t, b2)


def kernel(x, location_info, W1, b1, W2, b2, k):
    del x, k
    li = location_info
    pos = li[:, 6:8]
    idx = _knn_topk(pos)
    src = idx.reshape(-1)
    tgt = jnp.repeat(jnp.arange(N, dtype=jnp.int32), K)
    edge_index = jnp.stack([src, tgt], axis=0)

    li16 = jnp.pad(li, ((0, 0), (0, 6)))
    gs = _sc_gather(src, li16)
    gt = jnp.repeat(li16, K, axis=0)

    w1t = jnp.pad(jnp.transpose(W1), ((0, 4), (0, 0)))
    w2t = jnp.transpose(W2)
    edge_attr = _edge_mlp(gs, gt, w1t, b1.reshape(1, -1), w2t, b2.reshape(1, -1))
    return edge_index, edge_attr
